# baseline (speedup 1.0000x reference)
import jax
import jax.numpy as jnp
from jax.experimental import pallas as pl

N = 512
D = 128


def _fused_kernel(x_ref, gt_ref, W_ref, b_ref, w_row_ref, b_lin_ref,
                  loss_ref, g_ref):
    x = x_ref[...]
    W = W_ref[...]
    b = b_ref[...]
    w_row = w_row_ref[...]
    b_lin = b_lin_ref[0, 0]

    h = jnp.maximum(jnp.dot(x, W, preferred_element_type=jnp.float32) + b, 0.0)
    hw = h * w_row
    s = h * h

    c = jax.lax.dot_general(h, hw, (((1,), (1,)), ((), ())),
                            preferred_element_type=jnp.float32)
    a_col = jax.lax.dot_general(s, w_row, (((1,), (1,)), ((), ())),
                                preferred_element_type=jnp.float32)
    a_row = jax.lax.dot_general(w_row, s, (((1,), (1,)), ((), ())),
                                preferred_element_type=jnp.float32)

    d = a_col + a_row - 2.0 * c + b_lin
    g = jax.nn.sigmoid(d)
    g_ref[...] = g

    gt = gt_ref[...]
    p = jnp.clip(g, 1e-7, 1.0 - 1e-7)
    bce = gt * jnp.log(p) + (1.0 - gt) * jnp.log(1.0 - p)
    loss_ref[...] = -jnp.sum(bce, keepdims=True) / (N * N)


def kernel(x, grouping_matrix_true, W_embed, b_embed, w_lin, b_lin):
    w_row = w_lin.reshape(1, D)
    b2 = b_embed.reshape(1, D)
    bl2 = b_lin.reshape(1, 1)

    loss, g = pl.pallas_call(
        _fused_kernel,
        out_shape=(
            jax.ShapeDtypeStruct((1, 1), jnp.float32),
            jax.ShapeDtypeStruct((N, N), jnp.float32),
        ),
    )(x, grouping_matrix_true, W_embed, b2, w_row, bl2)

    h = jnp.maximum(x @ W_embed + b_embed, 0.0).astype(jnp.float32)
    diff = h[:, None, :] - h[None, :, :]
    dm = (diff * diff).reshape(N * N, -1)
    dv = (dm @ w_lin + b_lin).astype(jnp.float32)
    gm = jax.nn.sigmoid(dv).reshape(N, N)
    degree = jnp.sum(gm, axis=1)
    lap = jnp.diag(degree) - gm

    eigen_values, eigen_vectors = jnp.linalg.eigh(lap)
    sorted_indices = jnp.argsort(jnp.abs(eigen_values))
    fielder_value = eigen_values[sorted_indices[1]]
    fielder_vector = eigen_vectors[sorted_indices[1]]
    return loss[0, 0], fielder_value, fielder_vector, g
